# baseline (speedup 1.0000x reference)
import functools

import jax
import jax.numpy as jnp
from jax import lax
from jax.experimental import pallas as pl
from jax.experimental.pallas import tpu as pltpu
from jax.experimental.pallas import tpu_sc as plsc

_NC = 2
_NS = 16
_NW = _NC * _NS
_LANES = 16


def _make_kernel(B, L, D, V):
    bpw = B // _NW
    assert bpw == 128
    ch = 2
    nch = bpw // ch
    assert L % 8 == 0
    ltr = L // 8
    nh = 2
    lh = L // nh
    assert nh * lh == L and lh <= 128
    assert D == 2 * _LANES and V % 4 == 0

    mesh = plsc.VectorSubcoreMesh(core_axis_name="c", subcore_axis_name="s")

    @functools.partial(
        pl.kernel,
        mesh=mesh,
        out_type=jax.ShapeDtypeStruct((B, D), jnp.float32),
        scratch_types=[
            pltpu.VMEM((ltr, 8, 128), jnp.int32),
            pltpu.VMEM((bpw, nh, lh), jnp.int32),
            pltpu.VMEM((ch, nh, lh), jnp.int32),
            pltpu.VMEM((ch, L, 128), jnp.float32),
            pltpu.VMEM((ch, D), jnp.float32),
            pltpu.SemaphoreType.DMA,
        ],
        compiler_params=pltpu.CompilerParams(
            use_tc_tiling_on_sc=True, needs_layout_passes=False),
    )
    def k(idx4_hbm, table_hbm, out_hbm, idx_v, packed_v, shift_v, rows_v,
          out_v, sem):
        wid = lax.axis_index("s") * _NC + lax.axis_index("c")
        scale = jnp.float32(1.0 / L)
        lane = lax.iota(jnp.int32, 16)

        pltpu.sync_copy(idx4_hbm.at[:, wid], idx_v)

        groups = []
        for h in range(nh):
            offs = list(range(0, lh - 15, 16))
            if offs[-1] != lh - 16:
                offs.append(lh - 16)
            groups.extend((h, o) for o in offs)

        def repack_b(b, _):
            bcol = jnp.broadcast_to(b, (16,)).astype(jnp.int32)
            for h, o in groups:
                l = lane + (h * lh + o)
                v = plsc.load_gather(
                    idx_v, [l >> 3, jnp.bitwise_and(l, 7), bcol])
                packed_v[b, h, pl.ds(o, 16)] = v
            return _

        lax.fori_loop(0, bpw, repack_b, 0)

        def chunk_body(c, _):
            row0 = wid * bpw + c * ch
            for b in range(ch):
                for h, o in groups:
                    shift_v[b, h, pl.ds(o, 16)] = (
                        packed_v[c * ch + b, h, pl.ds(o, 16)] >> 2)
            copies = []
            for b in range(ch):
                for h in range(nh):
                    copies.append(pltpu.async_copy(
                        table_hbm.at[shift_v.at[b, h]],
                        rows_v.at[b, pl.ds(h * lh, lh)],
                        sem,
                    ))
            for cp in copies:
                cp.wait()
            red_groups = []
            for o in range(0, lh - 15, 16):
                red_groups.append((o, range(16)))
            rem = lh % 16
            if rem:
                red_groups.append((lh - 16, range(16 - rem, 16)))
            for b in range(ch):
                accs = [jnp.zeros((_LANES,), jnp.float32) for _ in range(4)]
                for h in range(nh):
                    for o, ks in red_groups:
                        svec = (packed_v[c * ch + b, h, pl.ds(o, 16)] & 3) * 32
                        for kk in ks:
                            s = svec[kk]
                            r = h * lh + o + kk
                            accs[2 * h] = accs[2 * h] + rows_v[
                                b, r, pl.ds(s, _LANES)]
                            accs[2 * h + 1] = accs[2 * h + 1] + rows_v[
                                b, r, pl.ds(s + _LANES, _LANES)]
                out_v[b, pl.ds(0, _LANES)] = (accs[0] + accs[2]) * scale
                out_v[b, pl.ds(_LANES, _LANES)] = (accs[1] + accs[3]) * scale
            pltpu.sync_copy(out_v, out_hbm.at[pl.ds(row0, ch)])
            return _

        lax.fori_loop(0, nch, chunk_body, 0)

    return k


def kernel(inputs, table):
    B, L = inputs.shape
    V, D = table.shape
    idx4 = (
        inputs.astype(jnp.int32)
        .T.reshape(L // 8, 8, B // 128, 128)
        .transpose(0, 2, 1, 3)
    )
    table128 = jnp.reshape(table, (V * D // 128, 128))
    return _make_kernel(B, L, D, V)(idx4, table128)


# speedup vs baseline: 1.3367x; 1.3367x over previous
import functools

import jax
import jax.numpy as jnp
from jax import lax
from jax.experimental import pallas as pl
from jax.experimental.pallas import tpu as pltpu
from jax.experimental.pallas import tpu_sc as plsc

_NC = 2
_NS = 16
_NW = _NC * _NS
_LANES = 16


def _tc_transpose_fold(table_t, V, D):
    cols = 4096
    grid = (V + cols - 1) // cols
    fold = 128 // D
    out_rows = cols // fold

    def body(xr, outr):
        y = xr[...].T
        outr[...] = jnp.concatenate(
            [y[j * out_rows:(j + 1) * out_rows] for j in range(fold)], axis=1)

    return pl.pallas_call(
        body,
        grid=(grid,),
        in_specs=[pl.BlockSpec((D, cols), lambda i: (0, i))],
        out_specs=pl.BlockSpec((out_rows, 128), lambda i: (i, 0)),
        out_shape=jax.ShapeDtypeStruct((grid * out_rows, 128), jnp.float32),
    )(table_t)


def _make_kernel(B, L, D, V):
    bpw = B // _NW
    assert bpw == 128
    ch = 2
    nch = bpw // ch
    assert L % 8 == 0
    ltr = L // 8
    nh = 2
    lh = L // nh
    assert nh * lh == L and lh <= 128
    assert D == 2 * _LANES and V % 4 == 0

    mesh = plsc.VectorSubcoreMesh(core_axis_name="c", subcore_axis_name="s")

    @functools.partial(
        pl.kernel,
        mesh=mesh,
        out_type=jax.ShapeDtypeStruct((B, D), jnp.float32),
        scratch_types=[
            pltpu.VMEM((ltr, 8, 128), jnp.int32),
            pltpu.VMEM((bpw, nh, lh), jnp.int32),
            pltpu.VMEM((ch, nh, lh), jnp.int32),
            pltpu.VMEM((ch, L, 128), jnp.float32),
            pltpu.VMEM((ch, D), jnp.float32),
            pltpu.SemaphoreType.DMA,
        ],
        compiler_params=pltpu.CompilerParams(
            use_tc_tiling_on_sc=True, needs_layout_passes=False),
    )
    def k(idx4_hbm, table_hbm, out_hbm, idx_v, packed_v, shift_v, rows_v,
          out_v, sem):
        wid = lax.axis_index("s") * _NC + lax.axis_index("c")
        scale = jnp.float32(1.0 / L)
        lane = lax.iota(jnp.int32, 16)

        pltpu.sync_copy(idx4_hbm.at[:, wid], idx_v)

        groups = []
        for h in range(nh):
            offs = list(range(0, lh - 15, 16))
            if offs[-1] != lh - 16:
                offs.append(lh - 16)
            groups.extend((h, o) for o in offs)

        def repack_b(b, _):
            bcol = jnp.broadcast_to(b, (16,)).astype(jnp.int32)
            for h, o in groups:
                l = lane + (h * lh + o)
                v = plsc.load_gather(
                    idx_v, [l >> 3, jnp.bitwise_and(l, 7), bcol])
                packed_v[b, h, pl.ds(o, 16)] = v
            return _

        lax.fori_loop(0, bpw, repack_b, 0)

        def chunk_body(c, _):
            row0 = wid * bpw + c * ch
            for b in range(ch):
                for h, o in groups:
                    v = packed_v[c * ch + b, h, pl.ds(o, 16)]
                    shift_v[b, h, pl.ds(o, 16)] = (
                        ((v >> 12) << 10) | (v & 1023))
            copies = []
            for b in range(ch):
                for h in range(nh):
                    copies.append(pltpu.async_copy(
                        table_hbm.at[shift_v.at[b, h]],
                        rows_v.at[b, pl.ds(h * lh, lh)],
                        sem,
                    ))
            for cp in copies:
                cp.wait()
            red_groups = []
            for o in range(0, lh - 15, 16):
                red_groups.append((o, range(16)))
            rem = lh % 16
            if rem:
                red_groups.append((lh - 16, range(16 - rem, 16)))
            for b in range(ch):
                accs = [jnp.zeros((_LANES,), jnp.float32) for _ in range(4)]
                for h in range(nh):
                    for o, ks in red_groups:
                        svec = ((packed_v[c * ch + b, h, pl.ds(o, 16)]
                                 >> 10) & 3) * 32
                        for kk in ks:
                            s = svec[kk]
                            r = h * lh + o + kk
                            accs[2 * h] = accs[2 * h] + rows_v[
                                b, r, pl.ds(s, _LANES)]
                            accs[2 * h + 1] = accs[2 * h + 1] + rows_v[
                                b, r, pl.ds(s + _LANES, _LANES)]
                out_v[b, pl.ds(0, _LANES)] = (accs[0] + accs[2]) * scale
                out_v[b, pl.ds(_LANES, _LANES)] = (accs[1] + accs[3]) * scale
            pltpu.sync_copy(out_v, out_hbm.at[pl.ds(row0, ch)])
            return _

        lax.fori_loop(0, nch, chunk_body, 0)

    return k


def kernel(inputs, table):
    B, L = inputs.shape
    V, D = table.shape
    idx4 = (
        inputs.astype(jnp.int32)
        .T.reshape(L // 8, 8, B // 128, 128)
        .transpose(0, 2, 1, 3)
    )
    table128 = _tc_transpose_fold(table.T, V, D)
    return _make_kernel(B, L, D, V)(idx4, table128)


# speedup vs baseline: 1.6753x; 1.2533x over previous
import functools

import jax
import jax.numpy as jnp
from jax import lax
from jax.experimental import pallas as pl
from jax.experimental.pallas import tpu as pltpu
from jax.experimental.pallas import tpu_sc as plsc

_NC = 2
_NS = 16
_NW = _NC * _NS
_LANES = 16


def _tc_transpose_fold(table_t, V, D):
    cols = 16384
    grid = (V + cols - 1) // cols
    fold = 128 // D
    out_rows = cols // fold

    def body(xr, outr):
        y = xr[...].T
        outr[...] = jnp.concatenate(
            [y[j * out_rows:(j + 1) * out_rows] for j in range(fold)], axis=1)

    return pl.pallas_call(
        body,
        grid=(grid,),
        in_specs=[pl.BlockSpec((D, cols), lambda i: (0, i))],
        out_specs=pl.BlockSpec((out_rows, 128), lambda i: (i, 0)),
        out_shape=jax.ShapeDtypeStruct((grid * out_rows, 128), jnp.float32),
        compiler_params=pltpu.CompilerParams(
            fuse_transposed_lhs_in_matmul=True),
    )(table_t)


def _make_kernel(B, L, D, V):
    bpw = B // _NW
    assert bpw == 128
    nchg = bpw // 2
    assert L % 8 == 0
    ltr = L // 8
    nh = 2
    lh = L // nh
    lh_pad = 8 * ((lh + 7) // 8)
    assert nh * lh == L and lh_pad <= 128
    assert D == 2 * _LANES and V % 4 == 0

    mesh = plsc.VectorSubcoreMesh(core_axis_name="c", subcore_axis_name="s")

    @functools.partial(
        pl.kernel,
        mesh=mesh,
        out_type=jax.ShapeDtypeStruct((B * D // 128, 128), jnp.float32),
        scratch_types=[
            pltpu.VMEM((ltr, 8, 128), jnp.int32),
            pltpu.VMEM((bpw, nh, lh_pad), jnp.int32),
            pltpu.VMEM((4, lh_pad), jnp.int32),
            pltpu.VMEM((4, lh_pad, 128), jnp.float32),
            pltpu.VMEM((bpw * D // 128, 128), jnp.float32),
            pltpu.SemaphoreType.DMA,
            pltpu.SemaphoreType.DMA,
            pltpu.SemaphoreType.DMA,
            pltpu.SemaphoreType.DMA,
        ],
        compiler_params=pltpu.CompilerParams(
            use_tc_tiling_on_sc=True, needs_layout_passes=False),
    )
    def k(idx4_hbm, table_hbm, out_hbm, idx_v, packed_v, shift_v, rows_v,
          out_v, sem0, sem1, sem2, sem3):
        wid = lax.axis_index("s") * _NC + lax.axis_index("c")
        scale = jnp.float32(1.0 / L)
        lane = lax.iota(jnp.int32, 16)

        pltpu.sync_copy(idx4_hbm.at[:, wid], idx_v)

        rp_offs = list(range(0, lh_pad - 15, 16))
        if rp_offs[-1] != lh_pad - 16:
            rp_offs.append(lh_pad - 16)
        groups = [(h, o) for h in range(nh) for o in rp_offs]

        def repack_b(b, _):
            bcol = jnp.broadcast_to(b, (16,)).astype(jnp.int32)
            for h, o in groups:
                l = jnp.minimum(lane + o, lh - 1) + h * lh
                v = plsc.load_gather(
                    idx_v, [l >> 3, jnp.bitwise_and(l, 7), bcol])
                packed_v[b, h, pl.ds(o, 16)] = v
            return _

        lax.fori_loop(0, bpw, repack_b, 0)

        sems = [sem0, sem1, sem2, sem3]
        red_groups = []
        for o in range(0, lh - 15, 16):
            red_groups.append((o, range(16)))
        rem = lh % 16
        if rem:
            red_groups.append((lh - 16, range(16 - rem, 16)))
        h_groups = [(h, o) for h in range(nh) for o, _ in red_groups]

        def fire(b, slot, h):
            for o in rp_offs:
                v = packed_v[b, h, pl.ds(o, 16)]
                shift_v[slot, pl.ds(o, 16)] = ((v >> 14) << 12) | (v & 4095)
            pltpu.async_copy(
                table_hbm.at[shift_v.at[slot]], rows_v.at[slot], sems[slot])

        def drain(slot):
            pltpu.make_async_copy(
                table_hbm.at[pl.ds(0, lh_pad)], rows_v.at[slot],
                sems[slot]).wait()

        for p in range(2):
            for h in range(nh):
                fire(jnp.int32(p), 2 * p + h, h)

        def pair_body(g, _):
            for p in range(2):
                b = 2 * g + p
                accs = [jnp.zeros((_LANES,), jnp.float32) for _ in range(4)]
                for h in range(nh):
                    slot = 2 * p + h
                    drain(slot)
                    for o, ks in red_groups:
                        svec = ((packed_v[b, h, pl.ds(o, 16)]
                                 >> 12) & 3) * 32
                        for kk in ks:
                            s = svec[kk]
                            accs[2 * h] = accs[2 * h] + rows_v[
                                slot, o + kk, pl.ds(s, _LANES)]
                            accs[2 * h + 1] = accs[2 * h + 1] + rows_v[
                                slot, o + kk, pl.ds(s + _LANES, _LANES)]
                    @pl.when(g < nchg - 1)
                    def _fire_next():
                        fire(b + 2, slot, h)
                od = (b & 3) * 32
                out_v[b >> 2, pl.ds(od, _LANES)] = (accs[0] + accs[2]) * scale
                out_v[b >> 2, pl.ds(od + _LANES, _LANES)] = (
                    (accs[1] + accs[3]) * scale)
            return _

        lax.fori_loop(0, nchg, pair_body, 0)
        orows = bpw * D // 128
        pltpu.sync_copy(out_v, out_hbm.at[pl.ds(wid * orows, orows)])

    return k


def kernel(inputs, table):
    B, L = inputs.shape
    V, D = table.shape
    idx4 = (
        inputs.astype(jnp.int32)
        .T.reshape(L // 8, 8, B // 128, 128)
        .transpose(0, 2, 1, 3)
    )
    table128 = _tc_transpose_fold(table.T, V, D)
    out128 = _make_kernel(B, L, D, V)(idx4, table128)
    return jnp.reshape(out128, (B, D))
